# speedup vs baseline: 7.5196x; 7.5196x over previous
import functools

import jax
import jax.numpy as jnp
from jax import lax
from jax.experimental import pallas as pl
from jax.experimental.pallas import tpu as pltpu
from jax.experimental.pallas import tpu_sc as plsc

B = 16384
NF = 9
VOCAB = 100000
DIM = 128

NC = 2
NS = 16
L = 16
NW = NC * NS
BPW = B // NW
C = 32
NCH = BPW // C
NCHG = B // C
DCH = DIM // L


def _make_kernel():
    mesh = plsc.VectorSubcoreMesh(core_axis_name="c", subcore_axis_name="s")

    @functools.partial(
        pl.kernel,
        mesh=mesh,
        out_type=jax.ShapeDtypeStruct((B, DIM), jnp.float32),
        scratch_types=[
            pltpu.VMEM((NF * NCH, C), jnp.int32),
            pltpu.VMEM((NF, L), jnp.float32),
            pltpu.VMEM((NF * C, DIM), jnp.float32),
            pltpu.VMEM((NF * C, DIM), jnp.float32),
            pltpu.VMEM((C, DIM), jnp.float32),
            pltpu.VMEM((C, DIM), jnp.float32),
            pltpu.SemaphoreType.DMA,
            pltpu.SemaphoreType.DMA,
            pltpu.SemaphoreType.DMA,
            pltpu.SemaphoreType.DMA,
        ],
    )
    def kern(data_c, tables, attn, out, idx_v, attn_v, rb0, rb1, os0, os1,
             sg0, sg1, so0, so1):
        wid = lax.axis_index("s") * NC + lax.axis_index("c")
        rbufs = (rb0, rb1)
        obufs = (os0, os1)
        gsems = (sg0, sg1)
        osems = (so0, so1)

        pltpu.sync_copy(attn, attn_v)
        for f in range(NF):
            pltpu.sync_copy(data_c.at[f, pl.ds(wid * NCH, NCH)],
                            idx_v.at[pl.ds(f * NCH, NCH)])

        def off_body(g, carry):
            for f in range(1, NF):
                for h in range(C // L):
                    sl = pl.ds(h * L, L)
                    idx_v[f * NCH + g, sl] = idx_v[f * NCH + g, sl] + (f * VOCAB)
            return carry
        lax.fori_loop(0, NCH, off_body, 0)

        w = [attn_v[f] for f in range(NF)]

        def issue_gathers(g, b):
            for f in range(NF):
                pltpu.async_copy(
                    tables.at[idx_v.at[f * NCH + g]],
                    rbufs[b].at[pl.ds(f * C, C)],
                    gsems[b],
                )

        def wait_gathers(b):
            for f in range(NF):
                pltpu.make_async_copy(
                    tables.at[idx_v.at[f * NCH]],
                    rbufs[b].at[pl.ds(f * C, C)],
                    gsems[b],
                ).wait()

        def compute(b):
            rb = rbufs[b]
            ob = obufs[b]

            def row_body(r, carry):
                for d in range(DCH):
                    sl = pl.ds(d * L, L)
                    acc = rb[r, sl] * w[0]
                    for f in range(1, NF):
                        acc = acc + rb[f * C + r, sl] * w[f]
                    ob[r, sl] = acc
                return carry
            lax.fori_loop(0, C, row_body, 0)

        def issue_out(g, b):
            pltpu.async_copy(obufs[b],
                             out.at[pl.ds((wid * NCH + g) * C, C)],
                             osems[b])

        def wait_out(b):
            pltpu.make_async_copy(obufs[b],
                                  out.at[pl.ds(wid * NCH * C, C)],
                                  osems[b]).wait()

        issue_gathers(0, 0)
        issue_gathers(1, 1)
        for g in (0, 1):
            b = g % 2
            wait_gathers(b)
            compute(b)
            issue_out(g, b)
            issue_gathers(g + 2, b)

        def chunk_pair(go, carry):
            for b in range(2):
                g = 2 * go + b
                wait_gathers(b)
                wait_out(b)
                compute(b)
                issue_out(g, b)
                issue_gathers(g + 2, b)
            return carry
        lax.fori_loop(1, NCH // 2 - 1, chunk_pair, 0)

        for g in (NCH - 2, NCH - 1):
            b = g % 2
            wait_gathers(b)
            wait_out(b)
            compute(b)
            issue_out(g, b)
        wait_out(0)
        wait_out(1)

    return kern


_kernel_fn = _make_kernel()


def kernel(data, tables, attn_score):
    data_c = jnp.transpose(data.astype(jnp.int32)).reshape(NF, NCHG, C)
    tables_flat = tables.reshape(NF * VOCAB, DIM)
    attn_b = jnp.broadcast_to(attn_score.astype(jnp.float32), (NF, L))
    out = _kernel_fn(data_c, tables_flat, attn_b)
    return (out, attn_score)
